# Initial kernel scaffold; baseline (speedup 1.0000x reference)
import jax
import jax.numpy as jnp
from jax.experimental import pallas as pl


def kernel(x, emb_table, fc1_w, fc1_b):
    raise NotImplementedError("write your pallas kernel here")


# speedup vs baseline: 12.3715x; 12.3715x over previous
import functools

import jax
import jax.numpy as jnp
from jax import lax
from jax.experimental import pallas as pl
from jax.experimental.pallas import tpu as pltpu
from jax.experimental.pallas import tpu_sc as plsc

NC = 2
NS = 16
NW = NC * NS
CH = 128


def _gather_call(x2d, emb_table, n_idx, hidden):
    chunks_per_w = n_idx // (NW * CH)
    rows_per_w = n_idx // NW
    mesh = plsc.VectorSubcoreMesh(core_axis_name="c", subcore_axis_name="s")

    @functools.partial(
        pl.kernel,
        mesh=mesh,
        out_type=jax.ShapeDtypeStruct((n_idx, hidden), jnp.float32),
        compiler_params=pltpu.CompilerParams(use_tc_tiling_on_sc=False),
        scratch_types=[
            pltpu.VMEM((chunks_per_w, CH), jnp.int32),
            pltpu.VMEM((2, CH, hidden), jnp.float32),
            pltpu.SemaphoreType.DMA,
        ],
    )
    def gather_k(x_hbm, table_hbm, out_hbm, idx_v, rows_v, gsem):
        wid = lax.axis_index("s") * NC + lax.axis_index("c")
        base = wid * rows_per_w
        pltpu.sync_copy(x_hbm.at[pl.ds(wid * chunks_per_w, chunks_per_w)], idx_v)
        pltpu.async_copy(table_hbm.at[idx_v.at[0]], rows_v.at[0], gsem)

        def body(j, carry):
            buf = lax.rem(j, 2)
            pltpu.make_async_copy(
                table_hbm.at[idx_v.at[j]], rows_v.at[buf], gsem
            ).wait()

            @pl.when(j + 1 < chunks_per_w)
            def _():
                pltpu.async_copy(
                    table_hbm.at[idx_v.at[j + 1]], rows_v.at[lax.rem(j + 1, 2)], gsem
                )

            pltpu.sync_copy(rows_v.at[buf], out_hbm.at[pl.ds(base + j * CH, CH)])
            return carry

        lax.fori_loop(0, chunks_per_w, body, 0)

    return gather_k(x2d, emb_table)


def _dense_body(e_ref, w_ref, b_ref, o_ref):
    e = e_ref[...]
    logits = jnp.dot(e, w_ref[...], preferred_element_type=jnp.float32) + b_ref[...]
    m = jnp.max(logits, axis=-1, keepdims=True)
    ez = jnp.exp(logits - m)
    lse = jnp.log(jnp.sum(ez, axis=-1, keepdims=True)) + m
    o_ref[...] = logits - lse


def _dense_call(embeds, wt, b2d, batch, hidden2, ncls):
    bb = 2048
    return pl.pallas_call(
        _dense_body,
        grid=(batch // bb,),
        in_specs=[
            pl.BlockSpec((bb, hidden2), lambda i: (i, 0)),
            pl.BlockSpec((hidden2, ncls), lambda i: (0, 0)),
            pl.BlockSpec((1, ncls), lambda i: (0, 0)),
        ],
        out_specs=pl.BlockSpec((bb, ncls), lambda i: (i, 0)),
        out_shape=jax.ShapeDtypeStruct((batch, ncls), jnp.float32),
    )(embeds, wt, b2d)


def kernel(x, emb_table, fc1_w, fc1_b):
    batch, hist = x.shape
    _, hidden = emb_table.shape
    ncls, hidden2 = fc1_w.shape
    n_idx = batch * hist
    x2d = x.reshape(n_idx // CH, CH).astype(jnp.int32)
    embeds = _gather_call(x2d, emb_table, n_idx, hidden)
    e2 = embeds.reshape(batch, hist * hidden)
    return _dense_call(e2, fc1_w.T, fc1_b.reshape(1, ncls), batch, hidden2, ncls)
